# baseline (speedup 1.0000x reference)
import functools
import math

import jax
import jax.numpy as jnp
from jax import lax
from jax.experimental import pallas as pl
from jax.experimental.pallas import tpu as pltpu, tpu_sc as plsc

H = 512
W = 512
N_ITERS = 5
NW = 32
OWN = H // NW
HALO = N_ITERS
EXT = OWN + 2 * HALO
NG = W // 16
PLANE = (EXT + 2) * W

_BREAKS = [float(1.0 / (1.0 + math.exp(k / 2.0))) for k in range(-4, 5)]

_mesh = plsc.VectorSubcoreMesh(core_axis_name="c", subcore_axis_name="s")


@functools.partial(
    pl.kernel,
    mesh=_mesh,
    out_type=jax.ShapeDtypeStruct((H * W,), jnp.float32),
    compiler_params=pltpu.CompilerParams(needs_layout_passes=False),
    scratch_types=[
        pltpu.VMEM((PLANE,), jnp.float32),
        pltpu.VMEM((PLANE,), jnp.float32),
        pltpu.VMEM((PLANE,), jnp.float32),
    ],
)
def _sc_icm(p_hbm, out_hbm, cpl, lab_a, lab_b):
    wid = lax.axis_index("s") * 2 + lax.axis_index("c")
    start = jnp.clip(wid * OWN - HALO, 0, H - EXT)
    off = wid * OWN - start + 1

    pltpu.sync_copy(p_hbm.at[pl.ds(start * W, EXT * W)], cpl.at[pl.ds(W, EXT * W)])

    lanes = lax.iota(jnp.int32, 16)
    zeros16 = jnp.zeros((16,), jnp.float32)
    ones16 = jnp.ones((16,), jnp.float32)
    halves16 = jnp.full((16,), 0.5, jnp.float32)

    for g in range(NG):
        lab_a[pl.ds(16 * g, 16)] = halves16
        lab_a[pl.ds((EXT + 1) * W + 16 * g, 16)] = halves16
        lab_b[pl.ds(16 * g, 16)] = halves16
        lab_b[pl.ds((EXT + 1) * W + 16 * g, 16)] = halves16

    def init_row(r, carry):
        base = lanes + r * W
        for g in range(NG):
            idx = base + (16 * g)
            p = plsc.load_gather(cpl, [idx])
            c = jnp.full((16,), -1.0, jnp.float32)
            for b in _BREAKS:
                c = c + jnp.where(p <= b, ones16, zeros16)
            lab0 = jnp.where(p > 0.5, ones16, zeros16)
            plsc.store_scatter(cpl, [idx], c)
            plsc.store_scatter(lab_a, [idx], lab0)
        return carry

    lax.fori_loop(1, EXT + 1, init_row, 0)

    lane_first = lanes == 0
    lane_last = lanes == 15
    zvec = jnp.zeros((16,), jnp.int32)

    def make_sweep(src, dst):
        def sweep_row(r, carry):
            base = lanes + r * W
            for g in range(NG):
                idx = base + (16 * g)
                up = plsc.load_gather(src, [idx - W])
                dn = plsc.load_gather(src, [idx + W])
                if g == 0:
                    il = jnp.where(lane_first, zvec, idx - 1)
                else:
                    il = idx - 1
                if g == NG - 1:
                    ir = jnp.where(lane_last, zvec, idx + 1)
                else:
                    ir = idx + 1
                lf = plsc.load_gather(src, [il])
                rt = plsc.load_gather(src, [ir])
                c = plsc.load_gather(cpl, [idx])
                s = (up + dn) + (lf + rt)
                lab = jnp.minimum(jnp.maximum((s + s) - c, zeros16), ones16)
                plsc.store_scatter(dst, [idx], lab)
            return carry

        return sweep_row

    planes = [lab_a, lab_b]
    for it in range(N_ITERS):
        src, dst = planes[it % 2], planes[(it + 1) % 2]
        lax.fori_loop(1, EXT + 1, make_sweep(src, dst), 0)

    final = planes[N_ITERS % 2]
    pltpu.sync_copy(
        final.at[pl.ds(off * W, OWN * W)],
        out_hbm.at[pl.ds(wid * (OWN * W), OWN * W)],
    )


@jax.jit
def kernel(probs):
    return _sc_icm(probs.reshape(H * W)).reshape(1, H, W)


# speedup vs baseline: 25.1711x; 25.1711x over previous
import functools

import jax
import jax.numpy as jnp
from jax.experimental import pallas as pl

GC_LAMBDA = 0.5
N_ITERS = 5
H = 512
W = 512


def _nsum(x, zero_row, zero_col):
    up = jnp.concatenate([x[1:, :], zero_row], axis=0)
    down = jnp.concatenate([zero_row, x[:-1, :]], axis=0)
    left = jnp.concatenate([x[:, 1:], zero_col], axis=1)
    right = jnp.concatenate([zero_col, x[:, :-1]], axis=1)
    return (up + down) + (left + right)


def _icm_kernel(p_ref, out_ref):
    p = p_ref[0]
    d2 = 2.0 * jnp.log((1.0 - p) / p)
    c = (jnp.clip(jnp.floor(d2), -5.0, 4.0) + 4.0).astype(jnp.bfloat16)
    half_row = jnp.full((1, W), 0.5, jnp.bfloat16)
    half_col = jnp.full((H, 1), 0.5, jnp.bfloat16)
    one = jnp.ones((H, W), jnp.bfloat16)
    zero = jnp.zeros((H, W), jnp.bfloat16)
    labels = (p > 0.5).astype(jnp.bfloat16)
    for _ in range(N_ITERS):
        s = _nsum(labels, half_row, half_col)
        labels = jnp.minimum(jnp.maximum((s + s) - c, zero), one)
    out_ref[0] = labels.astype(jnp.float32)


@jax.jit
def kernel(probs):
    return pl.pallas_call(
        _icm_kernel,
        out_shape=jax.ShapeDtypeStruct((1, H, W), jnp.float32),
    )(probs)


# speedup vs baseline: 26.0451x; 1.0347x over previous
import functools

import jax
import jax.numpy as jnp
from jax.experimental import pallas as pl
from jax.experimental.pallas import tpu as pltpu

GC_LAMBDA = 0.5
N_ITERS = 5
H = 512
W = 512
NB = 8
BR = H // NB


def _nsum(x, zero_row, zero_col):
    up = jnp.concatenate([x[1:, :], zero_row], axis=0)
    down = jnp.concatenate([zero_row, x[:-1, :]], axis=0)
    left = jnp.concatenate([x[:, 1:], zero_col], axis=1)
    right = jnp.concatenate([zero_col, x[:, :-1]], axis=1)
    return (up + down) + (left + right)


def _icm_kernel(p_hbm, out_hbm, p_vmem, out_vmem, c_ref, lab_ref, sem_in, sem_out):
    for b in range(NB):
        pltpu.make_async_copy(
            p_hbm.at[pl.ds(BR * b, BR)], p_vmem.at[pl.ds(BR * b, BR)],
            sem_in.at[b]).start()

    for b in range(NB):
        pltpu.make_async_copy(
            p_hbm.at[pl.ds(BR * b, BR)], p_vmem.at[pl.ds(BR * b, BR)],
            sem_in.at[b]).wait()
        p = p_vmem[pl.ds(BR * b, BR), :]
        d2 = 2.0 * jnp.log((1.0 - p) / p)
        c_ref[pl.ds(BR * b, BR), :] = (
            jnp.clip(jnp.floor(d2), -5.0, 4.0) + 4.0).astype(jnp.bfloat16)
        lab_ref[pl.ds(BR * b, BR), :] = (p > 0.5).astype(jnp.bfloat16)

    c = c_ref[...]
    half_row = jnp.full((1, W), 0.5, jnp.bfloat16)
    half_col = jnp.full((H, 1), 0.5, jnp.bfloat16)
    one = jnp.ones((H, W), jnp.bfloat16)
    zero = jnp.zeros((H, W), jnp.bfloat16)
    labels = lab_ref[...]
    for _ in range(N_ITERS - 1):
        s = _nsum(labels, half_row, half_col)
        labels = jnp.minimum(jnp.maximum((s + s) - c, zero), one)

    half_col_b = jnp.full((BR, 1), 0.5, jnp.bfloat16)
    for b in range(NB):
        r0 = BR * b
        xb = labels[r0:r0 + BR, :]
        if b == NB - 1:
            up = jnp.concatenate([labels[r0 + 1:, :], half_row], axis=0)
        else:
            up = labels[r0 + 1:r0 + BR + 1, :]
        if b == 0:
            down = jnp.concatenate([half_row, labels[:BR - 1, :]], axis=0)
        else:
            down = labels[r0 - 1:r0 + BR - 1, :]
        left = jnp.concatenate([xb[:, 1:], half_col_b], axis=1)
        right = jnp.concatenate([half_col_b, xb[:, :-1]], axis=1)
        s = (up + down) + (left + right)
        lab_b = jnp.minimum(jnp.maximum((s + s) - c[r0:r0 + BR, :],
                                        zero[:BR, :]), one[:BR, :])
        out_vmem[pl.ds(r0, BR), :] = lab_b.astype(jnp.float32)
        pltpu.make_async_copy(
            out_vmem.at[pl.ds(r0, BR)], out_hbm.at[pl.ds(r0, BR)],
            sem_out.at[b]).start()

    for b in range(NB):
        pltpu.make_async_copy(
            out_vmem.at[pl.ds(BR * b, BR)], out_hbm.at[pl.ds(BR * b, BR)],
            sem_out.at[b]).wait()


@jax.jit
def kernel(probs):
    out = pl.pallas_call(
        _icm_kernel,
        out_shape=jax.ShapeDtypeStruct((H, W), jnp.float32),
        in_specs=[pl.BlockSpec(memory_space=pl.ANY)],
        out_specs=pl.BlockSpec(memory_space=pl.ANY),
        scratch_shapes=[
            pltpu.VMEM((H, W), jnp.float32),
            pltpu.VMEM((H, W), jnp.float32),
            pltpu.VMEM((H, W), jnp.bfloat16),
            pltpu.VMEM((H, W), jnp.bfloat16),
            pltpu.SemaphoreType.DMA((NB,)),
            pltpu.SemaphoreType.DMA((NB,)),
        ],
    )(probs.reshape(H, W))
    return out.reshape(1, H, W)
